# Initial kernel scaffold; baseline (speedup 1.0000x reference)
import jax
import jax.numpy as jnp
from jax.experimental import pallas as pl


def kernel(sequence, position_ids, paper_ids, token_table, position_table, paper_table, embedding_weights, embedding_bias):
    raise NotImplementedError("write your pallas kernel here")


# speedup vs baseline: 1.7196x; 1.7196x over previous
import functools

import jax
import jax.numpy as jnp
from jax import lax
from jax.experimental import pallas as pl
from jax.experimental.pallas import tpu as pltpu
from jax.experimental.pallas import tpu_sc as plsc

B = 1024
S = 200
D = 64
N = B * S
NW = 32
PER_W = N // NW
GATHER = 128
CHUNK = 256
G_PER_CHUNK = CHUNK // GATHER
N_CHUNKS = PER_W // CHUNK
IDX_ROWS = PER_W // GATHER
LANES = 16
DBLK = D // LANES


def _sc_body(seq_hbm, pos_hbm, pap_hbm, tok_tab, pos_tab, pap_tab,
             w_hbm, b_hbm, out_hbm,
             idx_tok, idx_pos, idx_pap, buf_tok, buf_pos, buf_pap,
             w_v, b_v, sem):
    wid = lax.axis_index("s") * 2 + lax.axis_index("c")

    pltpu.sync_copy(seq_hbm.at[wid], idx_tok)
    pltpu.sync_copy(pos_hbm.at[wid], idx_pos)
    pltpu.sync_copy(pap_hbm.at[wid], idx_pap)
    pltpu.sync_copy(w_hbm, w_v)
    pltpu.sync_copy(b_hbm, b_v)

    w_tok = [w_v[0, pl.ds(j * LANES, LANES)] for j in range(DBLK)]
    w_pap = [w_v[1, pl.ds(j * LANES, LANES)] for j in range(DBLK)]
    w_pos = [w_v[2, pl.ds(j * LANES, LANES)] for j in range(DBLK)]
    bias = [b_v[pl.ds(j * LANES, LANES)] for j in range(DBLK)]

    def chunk_body(c, carry):
        copies = []
        for g in range(G_PER_CHUNK):
            r = c * G_PER_CHUNK + g
            dst = pl.ds(g * GATHER, GATHER)
            copies.append(pltpu.async_copy(
                tok_tab.at[idx_tok.at[r]], buf_tok.at[dst, :], sem))
            copies.append(pltpu.async_copy(
                pap_tab.at[idx_pap.at[r]], buf_pap.at[dst, :], sem))
            copies.append(pltpu.async_copy(
                pos_tab.at[idx_pos.at[r]], buf_pos.at[dst, :], sem))
        for cp in copies:
            cp.wait()

        def row_body(r, carry2):
            for j in range(DBLK):
                ds = pl.ds(j * LANES, LANES)
                acc = buf_tok[r, ds] * w_tok[j]
                acc += buf_pap[r, ds] * w_pap[j]
                acc += buf_pos[r, ds] * w_pos[j]
                buf_tok[r, ds] = acc + bias[j]
            return carry2

        lax.fori_loop(0, CHUNK, row_body, 0, unroll=2)
        pltpu.sync_copy(
            buf_tok, out_hbm.at[pl.ds(wid * PER_W + c * CHUNK, CHUNK), :])
        return carry

    lax.fori_loop(0, N_CHUNKS, chunk_body, 0)


def kernel(sequence, position_ids, paper_ids, token_table, position_table,
           paper_table, embedding_weights, embedding_bias):
    seq2d = sequence.reshape(NW, IDX_ROWS, GATHER).astype(jnp.int32)
    pos2d = position_ids.reshape(NW, IDX_ROWS, GATHER).astype(jnp.int32)
    pap2d = paper_ids.reshape(NW, IDX_ROWS, GATHER).astype(jnp.int32)

    mesh = plsc.VectorSubcoreMesh(core_axis_name="c", subcore_axis_name="s")
    run = functools.partial(
        pl.kernel,
        mesh=mesh,
        compiler_params=pltpu.CompilerParams(use_tc_tiling_on_sc=False),
        out_type=jax.ShapeDtypeStruct((N, D), jnp.float32),
        scratch_types=[
            pltpu.VMEM((IDX_ROWS, GATHER), jnp.int32),
            pltpu.VMEM((IDX_ROWS, GATHER), jnp.int32),
            pltpu.VMEM((IDX_ROWS, GATHER), jnp.int32),
            pltpu.VMEM((CHUNK, D), jnp.float32),
            pltpu.VMEM((CHUNK, D), jnp.float32),
            pltpu.VMEM((CHUNK, D), jnp.float32),
            pltpu.VMEM((3, D), jnp.float32),
            pltpu.VMEM((D,), jnp.float32),
            pltpu.SemaphoreType.DMA,
        ],
    )(_sc_body)
    out = run(seq2d, pos2d, pap2d, token_table, position_table, paper_table,
              embedding_weights, embedding_bias)
    return out.reshape(B, S, D)
